# Initial kernel scaffold; baseline (speedup 1.0000x reference)
import jax
import jax.numpy as jnp
from jax.experimental import pallas as pl


def kernel(x, hyperedge_index_2, hyperedge_type_2, A_2, C_w, C_b):
    raise NotImplementedError("write your pallas kernel here")


# speedup vs baseline: 13.2046x; 13.2046x over previous
import jax
import jax.numpy as jnp
from jax import lax
from jax.experimental import pallas as pl
from jax.experimental.pallas import tpu as pltpu
from jax.experimental.pallas import tpu_sc as plsc

N_NODES = 10000
D = 128
E_EDGES = 320000
T_TYPES = 4

NC = 2
NS = 16
L = 16
NW = NC * NS

C = 80
EW = E_EDGES // NW
E_PER_TILE = E_EDGES // NS
TN = T_TYPES * N_NODES
ROWS_A = 624
ROWS_LAST = N_NODES - (NS - 1) * ROWS_A


_BN1 = 2000


def _y_body(x_ref, at_ref, ab_ref, yt_ref, yb_ref):
    xb = x_ref[...]
    yt_ref[0] = jnp.dot(xb, at_ref[0], preferred_element_type=jnp.float32)
    yb_ref[0] = jnp.dot(xb, ab_ref[0], preferred_element_type=jnp.float32)


def _y_tables(x, a_top, a_bot):
    grid = (T_TYPES, N_NODES // _BN1)
    return pl.pallas_call(
        _y_body,
        grid=grid,
        in_specs=[
            pl.BlockSpec((_BN1, D), lambda t, i: (i, 0)),
            pl.BlockSpec((1, D, D), lambda t, i: (t, 0, 0)),
            pl.BlockSpec((1, D, D), lambda t, i: (t, 0, 0)),
        ],
        out_specs=[
            pl.BlockSpec((1, _BN1, D), lambda t, i: (t, i, 0)),
            pl.BlockSpec((1, _BN1, D), lambda t, i: (t, i, 0)),
        ],
        out_shape=[
            jax.ShapeDtypeStruct((T_TYPES, N_NODES, D), jnp.float32),
            jax.ShapeDtypeStruct((T_TYPES, N_NODES, D), jnp.float32),
        ],
    )(x, a_top, a_bot)


_BCAST_DNUMS = lax.GatherDimensionNumbers(
    offset_dims=(), collapsed_slice_dims=(0,), start_index_map=(0,))


def _sc_body(src0, src1, dst, typ, yt, yb, agg_out,
             s0_v, s1_v, dst_v, typ_v, g0_v, g1_v, tid_v, nrm_v,
             r0_v, r1_v, zbuf, ones_v, cnt_sh, agg_sh, sem0, sem1):
    cid = lax.axis_index("c")
    sid = lax.axis_index("s")
    wid = sid * NC + cid

    def _zero16(i, _):
        zbuf[pl.ds(i * L, L)] = jnp.zeros((L,), jnp.float32)
        return 0
    lax.fori_loop(0, 4000 // L, _zero16, 0)

    def _ones16(i, _):
        ones_v[pl.ds(i * L, L)] = jnp.full((L,), 1.0, jnp.float32)
        return 0
    lax.fori_loop(0, C // L, _ones16, 0)

    def _zrow(i, _):
        for j in range(D // L):
            r0_v[i, pl.ds(j * L, L)] = jnp.zeros((L,), jnp.float32)
        return 0
    lax.fori_loop(0, C, _zrow, 0)

    @pl.when(sid == 0)
    def _():
        for b in range(TN // 4000):
            pltpu.sync_copy(zbuf, cnt_sh.at[pl.ds(b * 4000, 4000)])

    row0 = sid * ROWS_A

    def _zero_agg_rows(base, nrows):
        off = 0
        while off < nrows:
            n = min(C, nrows - off)
            pltpu.sync_copy(r0_v.at[pl.ds(0, n), :],
                            agg_sh.at[pl.ds(base + off, n), :])
            off += n

    @pl.when(sid < NS - 1)
    def _():
        _zero_agg_rows(row0, ROWS_A)

    @pl.when(sid == NS - 1)
    def _():
        _zero_agg_rows(row0, ROWS_LAST)

    plsc.subcore_barrier()

    def _count_chunk(k, _):
        eb = sid * E_PER_TILE + k * C
        pltpu.sync_copy(dst.at[pl.ds(eb, C)], dst_v)
        pltpu.sync_copy(typ.at[pl.ds(eb, C)], typ_v)

        def _tid16(j, _):
            t16 = typ_v[pl.ds(j * L, L)]
            d16 = dst_v[pl.ds(j * L, L)]
            tid_v[pl.ds(j * L, L)] = t16 * N_NODES + d16
            return 0
        lax.fori_loop(0, C // L, _tid16, 0)
        pltpu.sync_copy(ones_v, cnt_sh.at[tid_v], add=True)
        return 0
    lax.fori_loop(0, E_PER_TILE // C, _count_chunk, 0)

    plsc.subcore_barrier()

    @pl.when(sid < TN // 4000)
    def _():
        base = sid * 4000
        pltpu.sync_copy(cnt_sh.at[pl.ds(base, 4000)], zbuf)

        def _nrm16(i, _):
            c16 = zbuf[pl.ds(i * L, L)]
            zbuf[pl.ds(i * L, L)] = 1.0 / jnp.maximum(c16, 1.0)
            return 0
        lax.fori_loop(0, 4000 // L, _nrm16, 0)
        pltpu.sync_copy(zbuf, cnt_sh.at[pl.ds(base, 4000)])

    plsc.subcore_barrier()

    def _edge_chunk(k, _):
        eb = wid * EW + k * C
        pltpu.sync_copy(src0.at[pl.ds(eb, C)], s0_v)
        pltpu.sync_copy(src1.at[pl.ds(eb, C)], s1_v)
        pltpu.sync_copy(dst.at[pl.ds(eb, C)], dst_v)
        pltpu.sync_copy(typ.at[pl.ds(eb, C)], typ_v)

        def _g16(j, _):
            t16 = typ_v[pl.ds(j * L, L)] * N_NODES
            g0_v[pl.ds(j * L, L)] = t16 + s0_v[pl.ds(j * L, L)]
            g1_v[pl.ds(j * L, L)] = t16 + s1_v[pl.ds(j * L, L)]
            tid_v[pl.ds(j * L, L)] = t16 + dst_v[pl.ds(j * L, L)]
            return 0
        lax.fori_loop(0, C // L, _g16, 0)

        pltpu.sync_copy(cnt_sh.at[tid_v], nrm_v)
        cp0 = pltpu.async_copy(yt.at[g0_v], r0_v, sem0)
        cp1 = pltpu.async_copy(yb.at[g1_v], r1_v, sem1)
        cp0.wait()
        cp1.wait()

        def _scale(jj, _):
            nrm16 = nrm_v[pl.ds(jj * L, L)]
            for i in range(L):
                e = jj * L + i
                nrm = lax.gather(
                    nrm16, jnp.full((L, 1), i, jnp.int32),
                    _BCAST_DNUMS, slice_sizes=(1,),
                    mode=lax.GatherScatterMode.PROMISE_IN_BOUNDS)
                for j in range(D // L):
                    sl = pl.ds(j * L, L)
                    r0_v[e, sl] = (r0_v[e, sl] + r1_v[e, sl]) * nrm
            return 0
        lax.fori_loop(0, C // L, _scale, 0)

        pltpu.sync_copy(r0_v, agg_sh.at[dst_v], add=True)
        return 0
    lax.fori_loop(0, EW // C, _edge_chunk, 0)

    plsc.subcore_barrier()

    @pl.when(sid < NS - 1)
    def _():
        pltpu.sync_copy(agg_sh.at[pl.ds(row0, ROWS_A), :],
                        agg_out.at[cid, pl.ds(row0, ROWS_A), :])

    @pl.when(sid == NS - 1)
    def _():
        pltpu.sync_copy(agg_sh.at[pl.ds(row0, ROWS_LAST), :],
                        agg_out.at[cid, pl.ds(row0, ROWS_LAST), :])


def _sc_scatter(src0, src1, dst, typ, yt, yb):
    mesh = plsc.VectorSubcoreMesh(core_axis_name="c", subcore_axis_name="s",
                                  num_cores=NC, num_subcores=NS)
    f = pl.kernel(
        _sc_body,
        out_type=jax.ShapeDtypeStruct((NC, N_NODES, D), jnp.float32),
        mesh=mesh,
        scratch_types=[
            pltpu.VMEM((C,), jnp.int32),
            pltpu.VMEM((C,), jnp.int32),
            pltpu.VMEM((C,), jnp.int32),
            pltpu.VMEM((C,), jnp.int32),
            pltpu.VMEM((C,), jnp.int32),
            pltpu.VMEM((C,), jnp.int32),
            pltpu.VMEM((C,), jnp.int32),
            pltpu.VMEM((C,), jnp.float32),
            pltpu.VMEM((C, D), jnp.float32),
            pltpu.VMEM((C, D), jnp.float32),
            pltpu.VMEM((4000,), jnp.float32),
            pltpu.VMEM((C,), jnp.float32),
            pltpu.VMEM_SHARED((TN,), jnp.float32),
            pltpu.VMEM_SHARED((N_NODES, D), jnp.float32),
            pltpu.SemaphoreType.DMA,
            pltpu.SemaphoreType.DMA,
        ],
    )
    return f(src0, src1, dst, typ, yt, yb)


_BN2 = 2000


def _out_body(x_ref, cwt_ref, cb_ref, a0_ref, a1_ref, o_ref):
    o_ref[...] = (jnp.dot(x_ref[...], cwt_ref[...],
                          preferred_element_type=jnp.float32)
                  + cb_ref[...] + a0_ref[...] + a1_ref[...])


def _combine(x, cwt, cb, a0, a1):
    grid = (N_NODES // _BN2,)
    return pl.pallas_call(
        _out_body,
        grid=grid,
        in_specs=[
            pl.BlockSpec((_BN2, D), lambda i: (i, 0)),
            pl.BlockSpec((D, D), lambda i: (0, 0)),
            pl.BlockSpec((1, D), lambda i: (0, 0)),
            pl.BlockSpec((_BN2, D), lambda i: (i, 0)),
            pl.BlockSpec((_BN2, D), lambda i: (i, 0)),
        ],
        out_specs=pl.BlockSpec((_BN2, D), lambda i: (i, 0)),
        out_shape=jax.ShapeDtypeStruct((N_NODES, D), jnp.float32),
    )(x, cwt, cb, a0, a1)


@jax.jit
def kernel(x, hyperedge_index_2, hyperedge_type_2, A_2, C_w, C_b):
    src = hyperedge_index_2[0]
    src0 = src[0::2]
    src1 = src[1::2]
    dst = hyperedge_index_2[1][0::2]
    a_top = A_2[:, :D, :]
    a_bot = A_2[:, D:, :]

    yt, yb = _y_tables(x, a_top, a_bot)
    yt = yt.reshape(T_TYPES * N_NODES, D)
    yb = yb.reshape(T_TYPES * N_NODES, D)

    agg = _sc_scatter(src0, src1, dst, hyperedge_type_2, yt, yb)

    return _combine(x, C_w.T, C_b.reshape(1, D), agg[0], agg[1])


# speedup vs baseline: 15.8076x; 1.1971x over previous
import jax
import jax.numpy as jnp
from jax import lax
from jax.experimental import pallas as pl
from jax.experimental.pallas import tpu as pltpu
from jax.experimental.pallas import tpu_sc as plsc

N_NODES = 10000
D = 128
E_EDGES = 320000
T_TYPES = 4

NC = 2
NS = 16
L = 16
NW = NC * NS

C = 80
EW = E_EDGES // NW
E_PER_TILE = E_EDGES // NS
TN = T_TYPES * N_NODES
ROWS_A = 624
ROWS_LAST = N_NODES - (NS - 1) * ROWS_A


_BN1 = 2000


def _y_body(x_ref, at_ref, ab_ref, yt_ref, yb_ref):
    xb = x_ref[...]
    yt_ref[0] = jnp.dot(xb, at_ref[0], preferred_element_type=jnp.float32)
    yb_ref[0] = jnp.dot(xb, ab_ref[0], preferred_element_type=jnp.float32)


def _y_tables(x, a_top, a_bot):
    grid = (T_TYPES, N_NODES // _BN1)
    return pl.pallas_call(
        _y_body,
        grid=grid,
        in_specs=[
            pl.BlockSpec((_BN1, D), lambda t, i: (i, 0)),
            pl.BlockSpec((1, D, D), lambda t, i: (t, 0, 0)),
            pl.BlockSpec((1, D, D), lambda t, i: (t, 0, 0)),
        ],
        out_specs=[
            pl.BlockSpec((1, _BN1, D), lambda t, i: (t, i, 0)),
            pl.BlockSpec((1, _BN1, D), lambda t, i: (t, i, 0)),
        ],
        out_shape=[
            jax.ShapeDtypeStruct((T_TYPES, N_NODES, D), jnp.float32),
            jax.ShapeDtypeStruct((T_TYPES, N_NODES, D), jnp.float32),
        ],
    )(x, a_top, a_bot)


_BCAST_DNUMS = lax.GatherDimensionNumbers(
    offset_dims=(), collapsed_slice_dims=(0,), start_index_map=(0,))


C1 = 400
NK1 = E_PER_TILE // C1
NK = EW // C
ZB = 2000


def _sc_body(src0, src1, dst, typ, yt, yb, agg_out,
             s0_v, s1_v, dst_v, typ_v, g0_v, g1_v, tid_v, nrm_v,
             r0_v, r1_v, d1_v, t1_v, tid1_v, ones1_v, zbuf,
             cnt_sh, agg_sh, semA0, semA1, semR0, semR1, semP0, semP1):
    semA = (semA0, semA1)
    semR = (semR0, semR1)
    semP = (semP0, semP1)
    cid = lax.axis_index("c")
    sid = lax.axis_index("s")
    wid = sid * NC + cid

    def _zero16(i, _):
        zbuf[pl.ds(i * L, L)] = jnp.zeros((L,), jnp.float32)
        return 0
    lax.fori_loop(0, ZB // L, _zero16, 0)

    def _ones16(i, _):
        ones1_v[pl.ds(i * L, L)] = jnp.full((L,), 1.0, jnp.float32)
        return 0
    lax.fori_loop(0, C1 // L, _ones16, 0)

    def _zrow(i, _):
        for j in range(D // L):
            r0_v[0][i, pl.ds(j * L, L)] = jnp.zeros((L,), jnp.float32)
        return 0
    lax.fori_loop(0, C, _zrow, 0)

    @pl.when(sid == 0)
    def _():
        for b in range(TN // ZB):
            pltpu.sync_copy(zbuf, cnt_sh.at[pl.ds(b * ZB, ZB)])

    row0 = sid * ROWS_A

    def _zero_agg_rows(base, nrows):
        off = 0
        while off < nrows:
            n = min(C, nrows - off)
            pltpu.sync_copy(r0_v[0].at[pl.ds(0, n), :],
                            agg_sh.at[pl.ds(base + off, n), :])
            off += n

    @pl.when(sid < NS - 1)
    def _():
        _zero_agg_rows(row0, ROWS_A)

    @pl.when(sid == NS - 1)
    def _():
        _zero_agg_rows(row0, ROWS_LAST)

    plsc.subcore_barrier()

    def _p1_slice(k):
        return pl.ds(sid * E_PER_TILE + k * C1, C1)

    def _p1_fire(k, b):
        @pl.when(k < NK1)
        def _():
            pltpu.async_copy(dst.at[_p1_slice(k)], d1_v[b], semP[b])
            pltpu.async_copy(typ.at[_p1_slice(k)], t1_v[b], semP[b])

    def _p1_proc(k, b):
        @pl.when(k < NK1)
        def _():
            pltpu.make_async_copy(dst.at[_p1_slice(k)], d1_v[b], semP[b]).wait()
            pltpu.make_async_copy(typ.at[_p1_slice(k)], t1_v[b], semP[b]).wait()

            def _tid16(j, _):
                t16 = t1_v[b][pl.ds(j * L, L)]
                d16 = d1_v[b][pl.ds(j * L, L)]
                tid1_v[pl.ds(j * L, L)] = t16 * N_NODES + d16
                return 0
            lax.fori_loop(0, C1 // L, _tid16, 0)
            pltpu.sync_copy(ones1_v, cnt_sh.at[tid1_v], add=True)

    _p1_fire(0, 0)
    _p1_fire(1, 1)

    def _p1_pair(i, _):
        k0 = 2 * i
        _p1_proc(k0, 0)
        _p1_fire(k0 + 2, 0)
        _p1_proc(k0 + 1, 1)
        _p1_fire(k0 + 3, 1)
        return 0
    lax.fori_loop(0, (NK1 + 1) // 2, _p1_pair, 0)

    plsc.subcore_barrier()

    def _to_norm(base):
        pltpu.sync_copy(cnt_sh.at[pl.ds(base, ZB)], zbuf)

        def _nrm16(i, _):
            c16 = zbuf[pl.ds(i * L, L)]
            zbuf[pl.ds(i * L, L)] = 1.0 / jnp.maximum(c16, 1.0)
            return 0
        lax.fori_loop(0, ZB // L, _nrm16, 0)
        pltpu.sync_copy(zbuf, cnt_sh.at[pl.ds(base, ZB)])

    _to_norm(sid * ZB)

    @pl.when(sid < TN // ZB - NS)
    def _():
        _to_norm((NS + sid) * ZB)

    plsc.subcore_barrier()

    def _p3_slice(k, n=C):
        return pl.ds(wid * EW + k * C, n)

    def _p3_fire_idx(k, b):
        @pl.when(k < NK)
        def _():
            pltpu.async_copy(src0.at[_p3_slice(k)], s0_v[b], semA[b])
            pltpu.async_copy(src1.at[_p3_slice(k)], s1_v[b], semA[b])
            pltpu.async_copy(dst.at[_p3_slice(k)], dst_v[b], semA[b])
            pltpu.async_copy(typ.at[_p3_slice(k)], typ_v[b], semA[b])

    def _p3_fire_rows(k, b):
        @pl.when(k < NK)
        def _():
            pltpu.make_async_copy(src0.at[_p3_slice(k)], s0_v[b], semA[b]).wait()
            pltpu.make_async_copy(src1.at[_p3_slice(k)], s1_v[b], semA[b]).wait()
            pltpu.make_async_copy(dst.at[_p3_slice(k)], dst_v[b], semA[b]).wait()
            pltpu.make_async_copy(typ.at[_p3_slice(k)], typ_v[b], semA[b]).wait()

            def _g16(j, _):
                t16 = typ_v[b][pl.ds(j * L, L)] * N_NODES
                g0_v[b][pl.ds(j * L, L)] = t16 + s0_v[b][pl.ds(j * L, L)]
                g1_v[b][pl.ds(j * L, L)] = t16 + s1_v[b][pl.ds(j * L, L)]
                tid_v[pl.ds(j * L, L)] = t16 + dst_v[b][pl.ds(j * L, L)]
                return 0
            lax.fori_loop(0, C // L, _g16, 0)

            pltpu.sync_copy(cnt_sh.at[tid_v], nrm_v[b])
            pltpu.async_copy(yt.at[g0_v[b]], r0_v[b], semR[b])
            pltpu.async_copy(yb.at[g1_v[b]], r1_v[b], semR[b])

    def _p3_proc(k, b):
        @pl.when(k < NK)
        def _():
            pltpu.make_async_copy(yt.at[g0_v[b]], r0_v[b], semR[b]).wait()
            pltpu.make_async_copy(yb.at[g1_v[b]], r1_v[b], semR[b]).wait()

            def _scale(jj, _):
                nrm16 = nrm_v[b][pl.ds(jj * L, L)]

                def _lane(i, _):
                    e = jj * L + i
                    nrm = lax.gather(
                        nrm16, jnp.full((L, 1), i, jnp.int32),
                        _BCAST_DNUMS, slice_sizes=(1,),
                        mode=lax.GatherScatterMode.PROMISE_IN_BOUNDS)
                    for j in range(D // L):
                        sl = pl.ds(j * L, L)
                        r0_v[b][e, sl] = (r0_v[b][e, sl] + r1_v[b][e, sl]) * nrm
                    return 0
                lax.fori_loop(0, L, _lane, 0)
                return 0
            lax.fori_loop(0, C // L, _scale, 0)

            pltpu.sync_copy(r0_v[b], agg_sh.at[dst_v[b]], add=True)

    _p3_fire_idx(0, 0)
    _p3_fire_rows(0, 0)
    _p3_fire_idx(1, 1)

    def _p3_pair(i, _):
        k0 = 2 * i
        _p3_fire_rows(k0 + 1, 1)
        _p3_proc(k0, 0)
        _p3_fire_idx(k0 + 2, 0)
        _p3_proc(k0 + 1, 1)
        _p3_fire_idx(k0 + 3, 1)
        _p3_fire_rows(k0 + 2, 0)
        return 0
    lax.fori_loop(0, (NK + 1) // 2, _p3_pair, 0)

    plsc.subcore_barrier()

    @pl.when(sid < NS - 1)
    def _():
        pltpu.sync_copy(agg_sh.at[pl.ds(row0, ROWS_A), :],
                        agg_out.at[cid, pl.ds(row0, ROWS_A), :])

    @pl.when(sid == NS - 1)
    def _():
        pltpu.sync_copy(agg_sh.at[pl.ds(row0, ROWS_LAST), :],
                        agg_out.at[cid, pl.ds(row0, ROWS_LAST), :])


def _sc_scatter(src0, src1, dst, typ, yt, yb):
    mesh = plsc.VectorSubcoreMesh(core_axis_name="c", subcore_axis_name="s",
                                  num_cores=NC, num_subcores=NS)
    f = pl.kernel(
        _sc_body,
        out_type=jax.ShapeDtypeStruct((NC, N_NODES, D), jnp.float32),
        mesh=mesh,
        scratch_types=[
            [pltpu.VMEM((C,), jnp.int32)] * 2,
            [pltpu.VMEM((C,), jnp.int32)] * 2,
            [pltpu.VMEM((C,), jnp.int32)] * 2,
            [pltpu.VMEM((C,), jnp.int32)] * 2,
            [pltpu.VMEM((C,), jnp.int32)] * 2,
            [pltpu.VMEM((C,), jnp.int32)] * 2,
            pltpu.VMEM((C,), jnp.int32),
            [pltpu.VMEM((C,), jnp.float32)] * 2,
            [pltpu.VMEM((C, D), jnp.float32)] * 2,
            [pltpu.VMEM((C, D), jnp.float32)] * 2,
            [pltpu.VMEM((C1,), jnp.int32)] * 2,
            [pltpu.VMEM((C1,), jnp.int32)] * 2,
            pltpu.VMEM((C1,), jnp.int32),
            pltpu.VMEM((C1,), jnp.float32),
            pltpu.VMEM((ZB,), jnp.float32),
            pltpu.VMEM_SHARED((TN,), jnp.float32),
            pltpu.VMEM_SHARED((N_NODES, D), jnp.float32),
            pltpu.SemaphoreType.DMA,
            pltpu.SemaphoreType.DMA,
            pltpu.SemaphoreType.DMA,
            pltpu.SemaphoreType.DMA,
            pltpu.SemaphoreType.DMA,
            pltpu.SemaphoreType.DMA,
        ],
    )
    return f(src0, src1, dst, typ, yt, yb)


_BN2 = 2000


def _out_body(x_ref, cwt_ref, cb_ref, a0_ref, a1_ref, o_ref):
    o_ref[...] = (jnp.dot(x_ref[...], cwt_ref[...],
                          preferred_element_type=jnp.float32)
                  + cb_ref[...] + a0_ref[...] + a1_ref[...])


def _combine(x, cwt, cb, a0, a1):
    grid = (N_NODES // _BN2,)
    return pl.pallas_call(
        _out_body,
        grid=grid,
        in_specs=[
            pl.BlockSpec((_BN2, D), lambda i: (i, 0)),
            pl.BlockSpec((D, D), lambda i: (0, 0)),
            pl.BlockSpec((1, D), lambda i: (0, 0)),
            pl.BlockSpec((_BN2, D), lambda i: (i, 0)),
            pl.BlockSpec((_BN2, D), lambda i: (i, 0)),
        ],
        out_specs=pl.BlockSpec((_BN2, D), lambda i: (i, 0)),
        out_shape=jax.ShapeDtypeStruct((N_NODES, D), jnp.float32),
    )(x, cwt, cb, a0, a1)


@jax.jit
def kernel(x, hyperedge_index_2, hyperedge_type_2, A_2, C_w, C_b):
    src = hyperedge_index_2[0]
    src0 = src[0::2]
    src1 = src[1::2]
    dst = hyperedge_index_2[1][0::2]
    a_top = A_2[:, :D, :]
    a_bot = A_2[:, D:, :]

    yt, yb = _y_tables(x, a_top, a_bot)
    yt = yt.reshape(T_TYPES * N_NODES, D)
    yb = yb.reshape(T_TYPES * N_NODES, D)

    agg = _sc_scatter(src0, src1, dst, hyperedge_type_2, yt, yb)

    return _combine(x, C_w.T, C_b.reshape(1, D), agg[0], agg[1])
